# baseline (speedup 1.0000x reference)
import functools

import jax
import jax.numpy as jnp
from jax import lax
from jax.experimental import pallas as pl
from jax.experimental import pallas as pl2
from jax.experimental.pallas import tpu as pltpu
from jax.experimental.pallas import tpu_sc as plsc

HIDDEN = 128
CHUNK = 128

_info = plsc.get_sparse_core_info()
_NC, _NS = _info.num_cores, _info.num_subcores
NW = _NC * _NS


def _make_gather(n_rows: int):
    n_per_w = n_rows // NW
    n_chunks = n_per_w // CHUNK

    mesh = plsc.VectorSubcoreMesh(core_axis_name="c", subcore_axis_name="s")

    @functools.partial(
        pl.kernel,
        mesh=mesh,
        out_type=jax.ShapeDtypeStruct((n_rows * HIDDEN,), jnp.float32),
        scratch_types=[
            pltpu.VMEM((n_chunks, CHUNK), jnp.int32),
            pltpu.VMEM_SHARED((_NS, CHUNK), jnp.int32),
            pltpu.SMEM((CHUNK,), jnp.int32),
            pltpu.VMEM_SHARED((_NS, 2, CHUNK * HIDDEN), jnp.float32),
            pltpu.SemaphoreType.DMA,
            pltpu.SemaphoreType.DMA,
        ],
    )
    def gather_kernel(idx_hbm, table_hbm, out_hbm, idx_v, idx_sp, idx_s, sp_v, rs, ws):
        cid = lax.axis_index("c")
        sid = lax.axis_index("s")
        wid = sid * _NC + cid
        pltpu.sync_copy(idx_hbm.at[pl.ds(wid * n_chunks, n_chunks)], idx_v)
        row_base = wid * n_per_w
        sp = sp_v.at[sid]

        def chunk_body(k, carry):
            pltpu.sync_copy(idx_v.at[k], idx_sp.at[sid])
            pltpu.sync_copy(idx_sp.at[sid], idx_s)
            bb = k % 2

            def row_body(r, carry2):
                i = idx_s[r]
                off = pl.multiple_of(i * HIDDEN, 8)
                pltpu.async_copy(
                    table_hbm.at[pl.ds(off, HIDDEN)],
                    sp.at[bb, pl.ds(r * HIDDEN, HIDDEN)],
                    rs,
                )
                return carry2

            lax.fori_loop(0, CHUNK, row_body, 0, unroll=False)
            pltpu.make_async_copy(
                table_hbm.at[pl.ds(0, CHUNK * HIDDEN)], sp.at[bb], rs
            ).wait()
            pltpu.sync_copy(
                sp.at[bb],
                out_hbm.at[pl.ds((row_base + k * CHUNK) * HIDDEN, CHUNK * HIDDEN)],
            )
            return carry

        lax.fori_loop(0, n_chunks, chunk_body, 0, unroll=False)

    return gather_kernel


def kernel(input_ids, weight):
    b, s = input_ids.shape
    n_rows = b * s
    idx = input_ids.reshape(n_rows // CHUNK, CHUNK).astype(jnp.int32)
    table_flat = weight.reshape(-1)
    out = _make_gather(n_rows)(idx, table_flat)
    return out.reshape(b, s, HIDDEN)
